# Initial kernel scaffold; baseline (speedup 1.0000x reference)
import jax
import jax.numpy as jnp
from jax.experimental import pallas as pl


def kernel(images):
    raise NotImplementedError("write your pallas kernel here")


# speedup vs baseline: 2.6804x; 2.6804x over previous
import functools

import jax
import jax.numpy as jnp
from jax import lax
from jax.experimental import pallas as pl
from jax.experimental.pallas import tpu as pltpu
from jax.experimental.pallas import tpu_sc as plsc

_NC = 2
_NS = 16
_NW = _NC * _NS
_L = 16
_B = 128
_N = 128
_M = 128
_BPW = _B // _NW
_CH = _M // _L


def _dp_body(img_hbm, out_hbm, theta_v, buf_a, buf_b, out_v):
    wid = lax.axis_index("s") * _NC + lax.axis_index("c")
    pltpu.sync_copy(img_hbm.at[wid], theta_v)

    iota = lax.iota(jnp.int32, _L)
    inf_v = jnp.full((_L,), jnp.inf, dtype=jnp.float32)
    buf_a[pl.ds(0, _L)] = inf_v
    buf_a[pl.ds(128, _L)] = inf_v
    buf_b[pl.ds(0, _L)] = inf_v
    buf_b[pl.ds(128, _L)] = inf_v

    for b in range(_BPW):
        def row_step(src, dst, i):
            for j in range(_CH):
                th = jnp.abs(theta_v[b, i, pl.ds(_L * j, _L)])
                lt = plsc.load_gather(src, [iota + (_L * j)])
                md = plsc.load_gather(src, [iota + (_L * j + 1)])
                rt = plsc.load_gather(src, [iota + (_L * j + 2)])
                cur = th + jnp.minimum(jnp.minimum(lt, md), rt)
                plsc.store_scatter(dst, [iota + (_L * j + 1)], cur)

        for j in range(_CH):
            th = jnp.abs(theta_v[b, 0, pl.ds(_L * j, _L)])
            plsc.store_scatter(buf_a, [iota + (_L * j + 1)], th)

        def pair(k, carry):
            row_step(buf_a, buf_b, 2 * k + 1)
            row_step(buf_b, buf_a, 2 * k + 2)
            return carry

        lax.fori_loop(0, (_N - 2) // 2, pair, jnp.int32(0))
        row_step(buf_a, buf_b, _N - 1)

        acc = plsc.load_gather(buf_b, [iota + 1])
        for j in range(1, _CH):
            acc = jnp.minimum(acc, plsc.load_gather(buf_b, [iota + (_L * j + 1)]))
        out_v[b, pl.ds(0, _L)] = jnp.full((_L,), jnp.min(acc), dtype=jnp.float32)

    pltpu.sync_copy(out_v, out_hbm.at[wid])


@jax.jit
def kernel(images):
    imgs = images.reshape(_NW, _BPW, _N, _M)
    run = functools.partial(
        pl.kernel,
        out_type=jax.ShapeDtypeStruct((_NW, _BPW, _L), jnp.float32),
        mesh=plsc.VectorSubcoreMesh(core_axis_name="c", subcore_axis_name="s"),
        scratch_types=[
            pltpu.VMEM((_BPW, _N, _M), jnp.float32),
            pltpu.VMEM((144,), jnp.float32),
            pltpu.VMEM((144,), jnp.float32),
            pltpu.VMEM((_BPW, _L), jnp.float32),
        ],
        compiler_params=pltpu.CompilerParams(needs_layout_passes=False),
    )(_dp_body)
    out = run(imgs)
    return out[:, :, 0].reshape(_B)


# speedup vs baseline: 4.1951x; 1.5651x over previous
import functools

import jax
import jax.numpy as jnp
from jax import lax
from jax.experimental import pallas as pl
from jax.experimental.pallas import tpu as pltpu
from jax.experimental.pallas import tpu_sc as plsc

_NC = 2
_NS = 16
_NW = _NC * _NS
_L = 16
_B = 128
_N = 128
_M = 128
_BPW = _B // _NW
_CH = _M // _L


def _lane_rot(v, idx):
    return jnp.take_along_axis(v, idx, axis=0, mode="promise_in_bounds")


def _dp_body(img_hbm, out_hbm, theta_v, out_v):
    wid = lax.axis_index("s") * _NC + lax.axis_index("c")
    pltpu.sync_copy(img_hbm.at[wid], theta_v)

    iota = lax.iota(jnp.int32, _L)
    inf_v = jnp.full((_L,), jnp.inf, dtype=jnp.float32)
    rotr_idx = jnp.bitwise_and(iota + (_L - 1), _L - 1)
    rotl_idx = jnp.bitwise_and(iota + 1, _L - 1)
    lane0 = iota == 0
    lane15 = iota == (_L - 1)
    offs = [iota * _CH + j for j in range(_CH)]

    for b in range(_BPW):
        def load_row(i):
            base = jnp.full((_L,), (b * _N + i) * _M, dtype=jnp.int32)
            return [
                jnp.abs(plsc.load_gather(theta_v, [base + offs[j]]))
                for j in range(_CH)
            ]

        def row_step(i, a):
            th = load_row(i)
            left0 = jnp.where(lane0, inf_v, _lane_rot(a[_CH - 1], rotr_idx))
            right7 = jnp.where(lane15, inf_v, _lane_rot(a[0], rotl_idx))
            new = []
            for j in range(_CH):
                lt = a[j - 1] if j > 0 else left0
                rt = a[j + 1] if j < _CH - 1 else right7
                new.append(th[j] + jnp.minimum(jnp.minimum(lt, a[j]), rt))
            return tuple(new)

        a_fin = lax.fori_loop(1, _N, row_step, tuple(load_row(0)))

        acc = a_fin[0]
        for j in range(1, _CH):
            acc = jnp.minimum(acc, a_fin[j])
        out_v[b, pl.ds(0, _L)] = jnp.full((_L,), jnp.min(acc), dtype=jnp.float32)

    pltpu.sync_copy(out_v, out_hbm.at[wid])


@jax.jit
def kernel(images):
    imgs = images.reshape(_NW, _BPW * _N * _M)
    run = functools.partial(
        pl.kernel,
        out_type=jax.ShapeDtypeStruct((_NW, _BPW, _L), jnp.float32),
        mesh=plsc.VectorSubcoreMesh(core_axis_name="c", subcore_axis_name="s"),
        scratch_types=[
            pltpu.VMEM((_BPW * _N * _M,), jnp.float32),
            pltpu.VMEM((_BPW, _L), jnp.float32),
        ],
        compiler_params=pltpu.CompilerParams(needs_layout_passes=False),
    )(_dp_body)
    out = run(imgs)
    return out[:, :, 0].reshape(_B)


# speedup vs baseline: 4.8590x; 1.1582x over previous
import functools

import jax
import jax.numpy as jnp
from jax import lax
from jax.experimental import pallas as pl
from jax.experimental.pallas import tpu as pltpu
from jax.experimental.pallas import tpu_sc as plsc

_NC = 2
_NS = 16
_NW = _NC * _NS
_L = 16
_B = 128
_N = 128
_M = 128
_BPW = _B // _NW
_CH = _M // _L


def _lane_rot(v, idx):
    return jnp.take_along_axis(v, idx, axis=0, mode="promise_in_bounds")


def _dp_body(img_hbm, out_hbm, theta_v, out_v, sems):
    wid = lax.axis_index("s") * _NC + lax.axis_index("c")
    slab = _N * _M
    copies = [
        pltpu.async_copy(
            img_hbm.at[pl.ds((wid * _BPW + b) * slab, slab)],
            theta_v.at[pl.ds(b * slab, slab)],
            sems.at[b],
        )
        for b in range(_BPW)
    ]

    iota = lax.iota(jnp.int32, _L)
    inf_v = jnp.full((_L,), jnp.inf, dtype=jnp.float32)
    rotr_idx = jnp.bitwise_and(iota + (_L - 1), _L - 1)
    rotl_idx = jnp.bitwise_and(iota + 1, _L - 1)
    lane0 = iota == 0
    lane15 = iota == (_L - 1)
    offs = [iota * _CH + j for j in range(_CH)]

    for b in range(_BPW):
        copies[b].wait()

        def load_row(i):
            base = jnp.full((_L,), (b * _N + i) * _M, dtype=jnp.int32)
            return [
                jnp.abs(plsc.load_gather(theta_v, [base + offs[j]]))
                for j in range(_CH)
            ]

        def row_step(i, a):
            th = load_row(i)
            left0 = jnp.where(lane0, inf_v, _lane_rot(a[_CH - 1], rotr_idx))
            right7 = jnp.where(lane15, inf_v, _lane_rot(a[0], rotl_idx))
            new = []
            for j in range(_CH):
                lt = a[j - 1] if j > 0 else left0
                rt = a[j + 1] if j < _CH - 1 else right7
                new.append(th[j] + jnp.minimum(jnp.minimum(lt, a[j]), rt))
            return tuple(new)

        def row_pair(k, a):
            return row_step(2 * k + 2, row_step(2 * k + 1, a))

        a_fin = lax.fori_loop(0, (_N - 2) // 2, row_pair, tuple(load_row(0)))
        a_fin = row_step(_N - 1, a_fin)

        acc = a_fin[0]
        for j in range(1, _CH):
            acc = jnp.minimum(acc, a_fin[j])
        out_v[b, pl.ds(0, _L)] = jnp.full((_L,), jnp.min(acc), dtype=jnp.float32)

    pltpu.sync_copy(out_v, out_hbm.at[wid])


@jax.jit
def kernel(images):
    imgs = images.reshape(_B * _N * _M)
    run = functools.partial(
        pl.kernel,
        out_type=jax.ShapeDtypeStruct((_NW, _BPW, _L), jnp.float32),
        mesh=plsc.VectorSubcoreMesh(core_axis_name="c", subcore_axis_name="s"),
        scratch_types=[
            pltpu.VMEM((_BPW * _N * _M,), jnp.float32),
            pltpu.VMEM((_BPW, _L), jnp.float32),
            pltpu.SemaphoreType.DMA((_BPW,)),
        ],
        compiler_params=pltpu.CompilerParams(needs_layout_passes=False),
    )(_dp_body)
    out = run(imgs)
    return out[:, :, 0].reshape(_B)


# speedup vs baseline: 4.8881x; 1.0060x over previous
import functools

import jax
import jax.numpy as jnp
from jax import lax
from jax.experimental import pallas as pl
from jax.experimental.pallas import tpu as pltpu
from jax.experimental.pallas import tpu_sc as plsc

_NC = 2
_NS = 16
_NW = _NC * _NS
_L = 16
_B = 128
_N = 128
_M = 128
_BPW = _B // _NW
_CH = _M // _L


def _lane_rot(v, idx):
    return jnp.take_along_axis(v, idx, axis=0, mode="promise_in_bounds")


def _dp_body(img_hbm, out_hbm, theta_v, out_v, sems):
    wid = lax.axis_index("s") * _NC + lax.axis_index("c")
    slab = _N * _M
    copies = [
        pltpu.async_copy(
            img_hbm.at[pl.ds((wid * _BPW + b) * slab, slab)],
            theta_v.at[pl.ds(b * slab, slab)],
            sems.at[b],
        )
        for b in range(_BPW)
    ]

    iota = lax.iota(jnp.int32, _L)
    inf_v = jnp.full((_L,), jnp.inf, dtype=jnp.float32)
    rotr_idx = jnp.bitwise_and(iota + (_L - 1), _L - 1)
    rotl_idx = jnp.bitwise_and(iota + 1, _L - 1)
    lane0 = iota == 0
    lane15 = iota == (_L - 1)
    offs = [iota * _CH + j for j in range(_CH)]

    for b in range(_BPW):
        copies[b].wait()

        def load_row(i):
            base = jnp.full((_L,), (b * _N + i) * _M, dtype=jnp.int32)
            return [
                jnp.abs(plsc.load_gather(theta_v, [base + offs[j]]))
                for j in range(_CH)
            ]

        def row_step(i, a):
            th = load_row(i)
            left0 = jnp.where(lane0, inf_v, _lane_rot(a[_CH - 1], rotr_idx))
            right7 = jnp.where(lane15, inf_v, _lane_rot(a[0], rotl_idx))
            new = []
            for j in range(_CH):
                lt = a[j - 1] if j > 0 else left0
                rt = a[j + 1] if j < _CH - 1 else right7
                new.append(th[j] + jnp.minimum(jnp.minimum(lt, a[j]), rt))
            return tuple(new)

        def row_pair(k, a):
            return row_step(2 * k + 2, row_step(2 * k + 1, a))

        a_fin = lax.fori_loop(0, (_N - 2) // 2, row_pair, tuple(load_row(0)))
        a_fin = row_step(_N - 1, a_fin)

        acc = a_fin[0]
        for j in range(1, _CH):
            acc = jnp.minimum(acc, a_fin[j])
        mn = jnp.full((_L,), jnp.min(acc), dtype=jnp.float32)
        plsc.store_scatter(out_v, [jnp.full((_L,), b, jnp.int32)], mn, mask=lane0)

    pltpu.sync_copy(out_v, out_hbm.at[wid])


@jax.jit
def kernel(images):
    imgs = images.reshape(_B * _N * _M)
    run = functools.partial(
        pl.kernel,
        out_type=jax.ShapeDtypeStruct((_NW, _BPW), jnp.float32),
        mesh=plsc.VectorSubcoreMesh(core_axis_name="c", subcore_axis_name="s"),
        scratch_types=[
            pltpu.VMEM((_BPW * _N * _M,), jnp.float32),
            pltpu.VMEM((_BPW,), jnp.float32),
            pltpu.SemaphoreType.DMA((_BPW,)),
        ],
        compiler_params=pltpu.CompilerParams(needs_layout_passes=False),
    )(_dp_body)
    out = run(imgs)
    return out.reshape(_B)


# speedup vs baseline: 4.9203x; 1.0066x over previous
import functools

import jax
import jax.numpy as jnp
from jax import lax
from jax.experimental import pallas as pl
from jax.experimental.pallas import tpu as pltpu
from jax.experimental.pallas import tpu_sc as plsc

_NC = 2
_NS = 16
_NW = _NC * _NS
_L = 16
_B = 128
_N = 128
_M = 128
_BPW = _B // _NW
_CH = _M // _L


def _lane_rot(v, idx):
    return jnp.take_along_axis(v, idx, axis=0, mode="promise_in_bounds")


def _dp_body(img_hbm, out_hbm, theta_v, out_v, sems):
    wid = lax.axis_index("s") * _NC + lax.axis_index("c")
    slab = _N * _M
    copies = [
        pltpu.async_copy(
            img_hbm.at[pl.ds((wid * _BPW + b) * slab, slab)],
            theta_v.at[pl.ds(b * slab, slab)],
            sems.at[b],
        )
        for b in range(_BPW)
    ]

    iota = lax.iota(jnp.int32, _L)
    inf_v = jnp.full((_L,), jnp.inf, dtype=jnp.float32)
    rotr_idx = jnp.bitwise_and(iota + (_L - 1), _L - 1)
    rotl_idx = jnp.bitwise_and(iota + 1, _L - 1)
    lane0 = iota == 0
    lane15 = iota == (_L - 1)
    offs = [iota * _CH + j for j in range(_CH)]

    for b in range(_BPW):
        copies[b].wait()

        def load_row(i):
            base = jnp.full((_L,), (b * _N + i) * _M, dtype=jnp.int32)
            return [
                jnp.abs(plsc.load_gather(theta_v, [base + offs[j]]))
                for j in range(_CH)
            ]

        def row_step(i, a):
            th = load_row(i)
            left0 = jnp.where(lane0, inf_v, _lane_rot(a[_CH - 1], rotr_idx))
            right7 = jnp.where(lane15, inf_v, _lane_rot(a[0], rotl_idx))
            new = []
            for j in range(_CH):
                lt = a[j - 1] if j > 0 else left0
                rt = a[j + 1] if j < _CH - 1 else right7
                new.append(th[j] + jnp.minimum(jnp.minimum(lt, a[j]), rt))
            return tuple(new)

        def row_pair(k, a):
            return row_step(2 * k + 2, row_step(2 * k + 1, a))

        a_fin = lax.fori_loop(0, (_N - 2) // 2, row_pair, tuple(load_row(0)))
        a_fin = row_step(_N - 1, a_fin)

        acc = a_fin[0]
        for j in range(1, _CH):
            acc = jnp.minimum(acc, a_fin[j])
        mn = jnp.full((_L,), jnp.min(acc), dtype=jnp.float32)
        plsc.store_scatter(out_v, [jnp.full((_L,), b, jnp.int32)], mn, mask=lane0)

    pltpu.sync_copy(out_v, out_hbm.at[wid])


@jax.jit
def kernel(images):
    imgs = images.reshape(_B * _N * _M)
    run = functools.partial(
        pl.kernel,
        out_type=jax.ShapeDtypeStruct((_NW, _BPW), jnp.float32),
        mesh=plsc.VectorSubcoreMesh(core_axis_name="c", subcore_axis_name="s"),
        scratch_types=[
            pltpu.VMEM((_BPW * _N * _M,), jnp.float32),
            pltpu.VMEM((_BPW,), jnp.float32),
            pltpu.SemaphoreType.DMA((_BPW,)),
        ],
        compiler_params=pltpu.CompilerParams(
            needs_layout_passes=False,
            disable_bounds_checks=True,
            disable_semaphore_checks=True,
        ),
    )(_dp_body)
    out = run(imgs)
    return out.reshape(_B)


# speedup vs baseline: 5.1958x; 1.0560x over previous
import functools

import jax
import jax.numpy as jnp
from jax import lax
from jax.experimental import pallas as pl
from jax.experimental.pallas import tpu as pltpu
from jax.experimental.pallas import tpu_sc as plsc

_NC = 2
_NS = 16
_NW = _NC * _NS
_L = 16
_B = 128
_N = 128
_M = 128
_BPW = _B // _NW
_CH = _M // _L


def _lane_rot(v, idx):
    return jnp.take_along_axis(v, idx, axis=0, mode="promise_in_bounds")


def _dp_body(img_hbm, out_hbm, theta_v, out_v, sems):
    wid = lax.axis_index("s") * _NC + lax.axis_index("c")
    slab = _N * _M
    copies = [
        pltpu.async_copy(
            img_hbm.at[pl.ds((wid * _BPW + b) * slab, slab)],
            theta_v.at[pl.ds(b * slab, slab)],
            sems.at[b],
        )
        for b in range(_BPW)
    ]

    iota = lax.iota(jnp.int32, _L)
    inf_v = jnp.full((_L,), jnp.inf, dtype=jnp.float32)
    rotr_idx = jnp.bitwise_and(iota + (_L - 1), _L - 1)
    rotl_idx = jnp.bitwise_and(iota + 1, _L - 1)
    lane0 = iota == 0
    lane15 = iota == (_L - 1)
    all_lanes = iota >= 0
    offs = [iota * _CH + j for j in range(_CH)]

    for b in range(_BPW):
        copies[b].wait()

        def load_row(i):
            row = theta_v.at[pl.ds((b * _N + i) * _M, _M)]
            return [
                jnp.abs(plsc.load_gather(row, [offs[j]], mask=all_lanes))
                for j in range(_CH)
            ]

        def row_step(i, a):
            th = load_row(i)
            left0 = jnp.where(lane0, inf_v, _lane_rot(a[_CH - 1], rotr_idx))
            right7 = jnp.where(lane15, inf_v, _lane_rot(a[0], rotl_idx))
            new = []
            for j in range(_CH):
                lt = a[j - 1] if j > 0 else left0
                rt = a[j + 1] if j < _CH - 1 else right7
                new.append(th[j] + jnp.minimum(jnp.minimum(lt, a[j]), rt))
            return tuple(new)

        def row_pair(k, a):
            return row_step(2 * k + 2, row_step(2 * k + 1, a))

        a_fin = lax.fori_loop(0, (_N - 2) // 2, row_pair, tuple(load_row(0)))
        a_fin = row_step(_N - 1, a_fin)

        acc = a_fin[0]
        for j in range(1, _CH):
            acc = jnp.minimum(acc, a_fin[j])
        mn = jnp.full((_L,), jnp.min(acc), dtype=jnp.float32)
        plsc.store_scatter(out_v, [jnp.full((_L,), b, jnp.int32)], mn, mask=lane0)

    pltpu.sync_copy(out_v, out_hbm.at[wid])


@jax.jit
def kernel(images):
    imgs = images.reshape(_B * _N * _M)
    run = functools.partial(
        pl.kernel,
        out_type=jax.ShapeDtypeStruct((_NW, _BPW), jnp.float32),
        mesh=plsc.VectorSubcoreMesh(core_axis_name="c", subcore_axis_name="s"),
        scratch_types=[
            pltpu.VMEM((_BPW * _N * _M,), jnp.float32),
            pltpu.VMEM((_BPW,), jnp.float32),
            pltpu.SemaphoreType.DMA((_BPW,)),
        ],
        compiler_params=pltpu.CompilerParams(
            needs_layout_passes=False,
            disable_bounds_checks=True,
            disable_semaphore_checks=True,
        ),
    )(_dp_body)
    out = run(imgs)
    return out.reshape(_B)
